# baseline (speedup 1.0000x reference)
import jax
import jax.numpy as jnp
from jax import lax
from jax.experimental import pallas as pl

TOKENS = 16384
HIDDEN = 4096
ROUTER = 64
EXPERTS = 64
BT = 512
NB = TOKENS // BT
NSTEP = NB // 2


def _top2_write(logits, idx_ref, w_ref):
    iota = lax.broadcasted_iota(jnp.int32, (BT, EXPERTS), 1)
    m1 = jnp.max(logits, axis=-1, keepdims=True)
    i1 = jnp.min(jnp.where(logits == m1, iota, EXPERTS), axis=-1, keepdims=True)
    masked = jnp.where(iota == i1, -jnp.inf, logits)
    m2 = jnp.max(masked, axis=-1, keepdims=True)
    i2 = jnp.min(jnp.where(masked == m2, iota, EXPERTS), axis=-1, keepdims=True)
    e = jnp.exp(m2 - m1)
    w1 = 1.0 / (1.0 + e)
    w2 = e / (1.0 + e)
    idx_ref[...] = jnp.concatenate([i1, i2], axis=1)
    w_ref[...] = jnp.concatenate([w1, w2], axis=1)


def _body(xa_ref, xb_ref, w_in_ref, b_in_ref, bm_ref, w_gate_ref, b_gate_ref,
          idx_a_ref, w_a_ref, idx_b_ref, w_b_ref):
    w_in = w_in_ref[...]
    bm = bm_ref[...]
    w_gate = w_gate_ref[...]
    b_in = b_in_ref[...]
    b_gate = b_gate_ref[...]

    def chain(x_blk):
        xp = jnp.dot(x_blk, w_in, preferred_element_type=jnp.float32) + b_in
        g = 0.1 * jnp.tanh(jnp.dot(xp, bm, preferred_element_type=jnp.float32))
        return jnp.dot(g, w_gate, preferred_element_type=jnp.float32) + b_gate

    _top2_write(chain(xa_ref[0]), idx_a_ref, w_a_ref)
    _top2_write(chain(xb_ref[0]), idx_b_ref, w_b_ref)


def kernel(x, W_in, b_in, tau, A, Bm, W_gate, b_gate):
    del tau, A
    xr = x.reshape(NB, BT, HIDDEN)
    b_in2 = b_in.reshape(1, ROUTER)
    b_gate2 = b_gate.reshape(1, EXPERTS)

    grid = (NSTEP,)
    half = TOKENS // 2
    idx_a, w_a, idx_b, w_b = pl.pallas_call(
        _body,
        grid=grid,
        in_specs=[
            pl.BlockSpec((1, BT, HIDDEN), lambda i: (i, 0, 0)),
            pl.BlockSpec((1, BT, HIDDEN), lambda i: (i + NSTEP, 0, 0)),
            pl.BlockSpec((HIDDEN, ROUTER), lambda i: (0, 0)),
            pl.BlockSpec((1, ROUTER), lambda i: (0, 0)),
            pl.BlockSpec((ROUTER, ROUTER), lambda i: (0, 0)),
            pl.BlockSpec((ROUTER, EXPERTS), lambda i: (0, 0)),
            pl.BlockSpec((1, EXPERTS), lambda i: (0, 0)),
        ],
        out_specs=[
            pl.BlockSpec((BT, 2), lambda i: (i, 0)),
            pl.BlockSpec((BT, 2), lambda i: (i, 0)),
            pl.BlockSpec((BT, 2), lambda i: (i, 0)),
            pl.BlockSpec((BT, 2), lambda i: (i, 0)),
        ],
        out_shape=[
            jax.ShapeDtypeStruct((half, 2), jnp.int32),
            jax.ShapeDtypeStruct((half, 2), jnp.float32),
            jax.ShapeDtypeStruct((half, 2), jnp.int32),
            jax.ShapeDtypeStruct((half, 2), jnp.float32),
        ],
    )(xr, xr, W_in, b_in2, Bm, W_gate, b_gate2)

    idx = jnp.concatenate([idx_a, idx_b], axis=0)
    w = jnp.concatenate([w_a, w_b], axis=0)
    return idx, w
